# baseline (speedup 1.0000x reference)
import functools

import jax
import jax.numpy as jnp
from jax import lax
from jax.experimental import pallas as pl
from jax.experimental.pallas import tpu as pltpu
from jax.experimental.pallas import tpu_sc as plsc

N = 10000
E = 160000
H = 256
HH = 128
B = 64
NP = 10240
_TWO48 = float(2 ** 48 - 1)

NBLK = 400
EBLK = 640

CK = 48
NCHUNK = 210
EPT = NCHUNK * CK
EPAD = 16 * EPT
NACC = 10112
RPT = NACC // 16


def _id_mlp_body(ids_ref, w1_ref, b1_ref, w2_ref, b2_ref, o_ref):
    ids = ids_ref[...].astype(jnp.float32)
    norm = jnp.clip((ids + 2.0) / _TWO48, 0.0, 1.0)
    h = jax.nn.relu(norm * w1_ref[...] + b1_ref[...])
    out = jnp.dot(h, w2_ref[...], preferred_element_type=jnp.float32) + b2_ref[...]
    o_ref[0, :, :] = out[:, :HH]
    o_ref[1, :, :] = out[:, HH:]


def _id_mlp(node_ids, w1, b1, w2, b2):
    return pl.pallas_call(
        _id_mlp_body,
        grid=(N // NBLK,),
        in_specs=[
            pl.BlockSpec((NBLK, 1), lambda i: (i, 0)),
            pl.BlockSpec((1, H), lambda i: (0, 0)),
            pl.BlockSpec((1, H), lambda i: (0, 0)),
            pl.BlockSpec((H, H), lambda i: (0, 0)),
            pl.BlockSpec((1, H), lambda i: (0, 0)),
        ],
        out_specs=pl.BlockSpec((2, NBLK, HH), lambda i: (0, i, 0)),
        out_shape=jax.ShapeDtypeStruct((2, NP, HH), jnp.float32),
    )(node_ids.reshape(N, 1), w1, b1.reshape(1, H), w2, b2.reshape(1, H))


def _edge_mlp_body(ea_ref, w1_ref, b1_ref, w2_ref, b2_ref, o_ref):
    h = jax.nn.relu(
        jnp.dot(ea_ref[...], w1_ref[...], preferred_element_type=jnp.float32)
        + b1_ref[...])
    out = jnp.dot(h, w2_ref[...], preferred_element_type=jnp.float32) + b2_ref[...]
    o_ref[0, :, :] = out[:, :HH]
    o_ref[1, :, :] = out[:, HH:]


def _edge_mlp(edge_attr, w1, b1, w2, b2):
    return pl.pallas_call(
        _edge_mlp_body,
        grid=(E // EBLK,),
        in_specs=[
            pl.BlockSpec((EBLK, 16), lambda i: (i, 0)),
            pl.BlockSpec((16, H), lambda i: (0, 0)),
            pl.BlockSpec((1, H), lambda i: (0, 0)),
            pl.BlockSpec((H, H), lambda i: (0, 0)),
            pl.BlockSpec((1, H), lambda i: (0, 0)),
        ],
        out_specs=pl.BlockSpec((2, EBLK, HH), lambda i: (0, i, 0)),
        out_shape=jax.ShapeDtypeStruct((2, EPAD, HH), jnp.float32),
    )(edge_attr, w1, b1.reshape(1, H), w2, b2.reshape(1, H))


def _conv_mlp_body(eps_ref, x_ref, agg_ref, w1_ref, b1_ref, w2_ref, b2_ref, o_ref):
    x = jnp.concatenate([x_ref[0], x_ref[1]], axis=1)
    agg = jnp.concatenate([agg_ref[0], agg_ref[1]], axis=1)
    y = eps_ref[0, 0] * x + agg
    h = jax.nn.relu(jnp.dot(y, w1_ref[...], preferred_element_type=jnp.float32)
                    + b1_ref[...])
    out = jax.nn.relu(
        jnp.dot(h, w2_ref[...], preferred_element_type=jnp.float32) + b2_ref[...])
    o_ref[0, :, :] = out[:, :HH]
    o_ref[1, :, :] = out[:, HH:]


def _conv_mlp(eps1p, xt, agg, w1, b1, w2, b2):
    return pl.pallas_call(
        _conv_mlp_body,
        grid=(N // NBLK,),
        in_specs=[
            pl.BlockSpec(memory_space=pltpu.SMEM),
            pl.BlockSpec((2, NBLK, HH), lambda i: (0, i, 0)),
            pl.BlockSpec((2, NBLK, HH), lambda i: (0, i, 0)),
            pl.BlockSpec((H, H), lambda i: (0, 0)),
            pl.BlockSpec((1, H), lambda i: (0, 0)),
            pl.BlockSpec((H, H), lambda i: (0, 0)),
            pl.BlockSpec((1, H), lambda i: (0, 0)),
        ],
        out_specs=pl.BlockSpec((2, NBLK, HH), lambda i: (0, i, 0)),
        out_shape=jax.ShapeDtypeStruct((2, NP, HH), jnp.float32),
    )(eps1p, xt, agg, w1, b1.reshape(1, H), w2, b2.reshape(1, H))


def _conv_mlp_pool_body(eps_ref, x_ref, agg_ref, w1_ref, b1_ref, w2_ref, b2_ref,
                        batch_ref, sum_ref, cnt_ref):
    x = jnp.concatenate([x_ref[0], x_ref[1]], axis=1)
    agg = jnp.concatenate([agg_ref[0], agg_ref[1]], axis=1)
    y = eps_ref[0, 0] * x + agg
    h = jax.nn.relu(jnp.dot(y, w1_ref[...], preferred_element_type=jnp.float32)
                    + b1_ref[...])
    out = jax.nn.relu(
        jnp.dot(h, w2_ref[...], preferred_element_type=jnp.float32) + b2_ref[...])
    seg = jax.lax.broadcasted_iota(jnp.int32, (1, B), 1)
    onehot = (batch_ref[...] == seg).astype(jnp.float32)
    psum = lax.dot_general(onehot, out, (((0,), (0,)), ((), ())),
                           preferred_element_type=jnp.float32,
                           precision=lax.Precision.HIGHEST)
    pcnt = lax.dot_general(onehot, jnp.ones((NBLK, HH), jnp.float32),
                           (((0,), (0,)), ((), ())),
                           preferred_element_type=jnp.float32,
                           precision=lax.Precision.HIGHEST)

    @pl.when(pl.program_id(0) == 0)
    def _():
        sum_ref[...] = jnp.zeros_like(sum_ref)
        cnt_ref[...] = jnp.zeros_like(cnt_ref)

    sum_ref[...] += psum
    cnt_ref[...] += pcnt


def _conv_mlp_pool(eps1p, xt, agg, w1, b1, w2, b2, batch):
    return pl.pallas_call(
        _conv_mlp_pool_body,
        grid=(N // NBLK,),
        in_specs=[
            pl.BlockSpec(memory_space=pltpu.SMEM),
            pl.BlockSpec((2, NBLK, HH), lambda i: (0, i, 0)),
            pl.BlockSpec((2, NBLK, HH), lambda i: (0, i, 0)),
            pl.BlockSpec((H, H), lambda i: (0, 0)),
            pl.BlockSpec((1, H), lambda i: (0, 0)),
            pl.BlockSpec((H, H), lambda i: (0, 0)),
            pl.BlockSpec((1, H), lambda i: (0, 0)),
            pl.BlockSpec((NBLK, 1), lambda i: (i, 0)),
        ],
        out_specs=[
            pl.BlockSpec((B, H), lambda i: (0, 0)),
            pl.BlockSpec((B, HH), lambda i: (0, 0)),
        ],
        out_shape=[
            jax.ShapeDtypeStruct((B, H), jnp.float32),
            jax.ShapeDtypeStruct((B, HH), jnp.float32),
        ],
    )(eps1p, xt, agg, w1, b1.reshape(1, H), w2, b2.reshape(1, H),
      batch.reshape(N, 1))


def _head_body(ss_ref, sc_ref, gs_ref, gc_ref, dep_ref, w1s_ref, w1g_ref,
               w1d_ref, b1_ref, w2_ref, b2_ref, o_ref):
    s_emb = ss_ref[...] / jnp.maximum(sc_ref[:, :1], 1.0)
    g_emb = gs_ref[...] / jnp.maximum(gc_ref[:, :1], 1.0)
    dep = dep_ref[...].astype(jnp.bfloat16).astype(jnp.float32)
    w1d = w1d_ref[...].astype(jnp.bfloat16).astype(jnp.float32)
    z = (jnp.dot(s_emb, w1s_ref[...], preferred_element_type=jnp.float32)
         + jnp.dot(g_emb, w1g_ref[...], preferred_element_type=jnp.float32)
         + dep * w1d
         + b1_ref[...])
    h = jax.nn.relu(z)
    o_ref[...] = (jnp.dot(h, w2_ref[...], preferred_element_type=jnp.float32)
                  + b2_ref[...])


def _head(s_sum, s_cnt, g_sum, g_cnt, depth, w1, b1, w2, b2):
    out = pl.pallas_call(
        _head_body,
        out_shape=jax.ShapeDtypeStruct((B, 1), jnp.float32),
    )(s_sum, s_cnt, g_sum, g_cnt, depth.reshape(B, 1),
      w1[:H], w1[H:2 * H], w1[2 * H:].reshape(1, H), b1.reshape(1, H),
      w2, b2.reshape(1, 1))
    return out[:, 0]


def _msg_body(xt_ref, et_ref, ei_ref, zr_ref, out_ref,
              xbA, xbB, ebA, ebB, mbA, mbB, ixA, ixB, sxA, sxB,
              acc, gsA, gsB, esA, esB, ssA, ssB, imA, imB):
    cid = lax.axis_index("c")
    sid = lax.axis_index("s")
    rbase = sid * RPT
    pltpu.sync_copy(zr_ref, acc.at[pl.ds(rbase, RPT)])
    plsc.subcore_barrier()

    ibase = sid * EPT
    ebase = cid * EPAD + sid * EPT
    off = cid * NP

    gbase = sid * NCHUNK

    def idx_load(g, ix, im):
        pltpu.async_copy(ei_ref.at[:, gbase + g], ix, im)

    def idx_wait(g, ix, im):
        pltpu.make_async_copy(ei_ref.at[:, gbase + g], ix, im).wait()
        for i in range(CK // 16):
            s = pl.ds(i * 16, 16)
            ix[0, s] = ix[0, s] + off

    def loads(g, ix, xb, eb, gs, es):
        pltpu.async_copy(xt_ref.at[ix.at[0]], xb, gs)
        pltpu.async_copy(et_ref.at[pl.ds(ebase + g * CK, CK)], eb, es)

    def proc(g, xb, eb, mb, ix, sx, gs, es, ss, im, steady):
        pltpu.make_async_copy(xt_ref.at[ix.at[0]], xb, gs).wait()
        pltpu.make_async_copy(et_ref.at[pl.ds(ebase + g * CK, CK)], eb,
                              es).wait()
        if steady:
            pltpu.make_async_copy(mb, acc.at[sx], ss).wait()
        for i in range(CK // 16):
            s = pl.ds(i * 16, 16)
            sx[s] = ix[1, s]

        @pl.when(g + 2 < NCHUNK)
        def _():
            idx_load(g + 2, ix, im)

        def rowop(r, c):
            for j in range(HH // 16):
                s = pl.ds(j * 16, 16)
                mb[r, s] = jnp.maximum(xb[r, s] + eb[r, s], 0.0)
            return c

        lax.fori_loop(0, CK, rowop, 0)
        pltpu.async_copy(mb, acc.at[sx], ss, add=True)

        @pl.when(g + 2 < NCHUNK)
        def _():
            idx_wait(g + 2, ix, im)
            loads(g + 2, ix, xb, eb, gs, es)

    idx_load(0, ixA, imA)
    idx_wait(0, ixA, imA)
    loads(0, ixA, xbA, ebA, gsA, esA)
    idx_load(1, ixB, imB)
    idx_wait(1, ixB, imB)
    loads(1, ixB, xbB, ebB, gsB, esB)

    proc(0, xbA, ebA, mbA, ixA, sxA, gsA, esA, ssA, imA, False)
    proc(1, xbB, ebB, mbB, ixB, sxB, gsB, esB, ssB, imB, False)

    def pair(i, c):
        g = 2 * i + 2
        proc(g, xbA, ebA, mbA, ixA, sxA, gsA, esA, ssA, imA, True)
        proc(g + 1, xbB, ebB, mbB, ixB, sxB, gsB, esB, ssB, imB, True)
        return c

    lax.fori_loop(0, (NCHUNK - 2) // 2, pair, 0)
    pltpu.make_async_copy(mbA, acc.at[sxA], ssA).wait()
    pltpu.make_async_copy(mbB, acc.at[sxB], ssB).wait()
    plsc.subcore_barrier()
    pltpu.sync_copy(acc.at[pl.ds(rbase, RPT)],
                    out_ref.at[cid].at[pl.ds(rbase, RPT)])


@functools.cache
def _get_msg_kernel():
    return pl.kernel(
        _msg_body,
        out_type=jax.ShapeDtypeStruct((2, NACC, HH), jnp.float32),
        mesh=plsc.VectorSubcoreMesh(core_axis_name="c", subcore_axis_name="s",
                                    num_cores=2, num_subcores=16),
        scratch_types=[
            pltpu.VMEM((CK, HH), jnp.float32),
            pltpu.VMEM((CK, HH), jnp.float32),
            pltpu.VMEM((CK, HH), jnp.float32),
            pltpu.VMEM((CK, HH), jnp.float32),
            pltpu.VMEM((CK, HH), jnp.float32),
            pltpu.VMEM((CK, HH), jnp.float32),
            pltpu.VMEM((2, CK), jnp.int32),
            pltpu.VMEM((2, CK), jnp.int32),
            pltpu.VMEM((CK,), jnp.int32),
            pltpu.VMEM((CK,), jnp.int32),
            pltpu.VMEM_SHARED((NACC, HH), jnp.float32),
            pltpu.SemaphoreType.DMA,
            pltpu.SemaphoreType.DMA,
            pltpu.SemaphoreType.DMA,
            pltpu.SemaphoreType.DMA,
            pltpu.SemaphoreType.DMA,
            pltpu.SemaphoreType.DMA,
            pltpu.SemaphoreType.DMA,
            pltpu.SemaphoreType.DMA,
        ],
    )


def _message_pass(xt, et, ei_p, zrows):
    return _get_msg_kernel()(xt.reshape(2 * NP, HH), et.reshape(2 * EPAD, HH),
                             ei_p.reshape(2, 16 * NCHUNK, CK), zrows)


def _encode(node_ids, edge_index, e_emb, batch, params, p1, p2, zrows):
    pad = jnp.zeros((2, EPAD - E), jnp.int32).at[1].set(N)
    ei_p = jnp.concatenate([edge_index.astype(jnp.int32), pad], axis=1)
    xt = _id_mlp(node_ids, params['id_W1'], params['id_b1'],
                 params['id_W2'], params['id_b2'])
    agg1 = _message_pass(xt, e_emb, ei_p, zrows)
    eps1 = (1.0 + params[p1 + '_eps']).astype(jnp.float32).reshape(1, 1)
    xt = _conv_mlp(eps1, xt, agg1, params[p1 + '_W1'], params[p1 + '_b1'],
                   params[p1 + '_W2'], params[p1 + '_b2'])
    agg2 = _message_pass(xt, e_emb, ei_p, zrows)
    eps2 = (1.0 + params[p2 + '_eps']).astype(jnp.float32).reshape(1, 1)
    return _conv_mlp_pool(eps2, xt, agg2, params[p2 + '_W1'], params[p2 + '_b1'],
                          params[p2 + '_W2'], params[p2 + '_b2'],
                          batch.astype(jnp.int32))


def kernel(s_node_ids, s_edge_index, s_edge_attr, s_batch, depth,
           g_node_ids, g_edge_index, g_edge_attr, g_batch, params):
    zrows = jnp.zeros((RPT, HH), jnp.float32)
    s_e = _edge_mlp(s_edge_attr.astype(jnp.float32), params['edge_W1'],
                    params['edge_b1'], params['edge_W2'], params['edge_b2'])
    g_e = _edge_mlp(g_edge_attr.astype(jnp.float32), params['edge_W1'],
                    params['edge_b1'], params['edge_W2'], params['edge_b2'])
    s_sum, s_cnt = _encode(s_node_ids, s_edge_index, s_e, s_batch, params,
                           's1', 's2', zrows)
    g_sum, g_cnt = _encode(g_node_ids, g_edge_index, g_e, g_batch, params,
                           'g1', 'g2', zrows)
    return _head(s_sum, s_cnt, g_sum, g_cnt, depth.astype(jnp.float32),
                 params['reg_W1'], params['reg_b1'], params['reg_W2'],
                 params['reg_b2'])


# speedup vs baseline: 1.0610x; 1.0610x over previous
import functools

import jax
import jax.numpy as jnp
from jax import lax
from jax.experimental import pallas as pl
from jax.experimental.pallas import tpu as pltpu
from jax.experimental.pallas import tpu_sc as plsc

N = 10000
E = 160000
H = 256
HH = 128
B = 64
NP = 10240
_TWO48 = float(2 ** 48 - 1)

NBLK = 400
EBLK = 640

CK = 48
NCHUNK = 210
EPT = NCHUNK * CK
EPAD = 16 * EPT
NACC = 10112
RPT = NACC // 16


def _id_mlp_body(ids_ref, w1_ref, b1_ref, w2_ref, b2_ref, o_ref):
    ids = ids_ref[...].astype(jnp.float32)
    norm = jnp.clip((ids + 2.0) / _TWO48, 0.0, 1.0)
    h = jax.nn.relu(norm * w1_ref[...] + b1_ref[...])
    out = jnp.dot(h, w2_ref[...], preferred_element_type=jnp.float32) + b2_ref[...]
    o_ref[0, :, :] = out[:, :HH]
    o_ref[1, :, :] = out[:, HH:]


def _id_mlp(node_ids, w1, b1, w2, b2):
    return pl.pallas_call(
        _id_mlp_body,
        grid=(N // NBLK,),
        in_specs=[
            pl.BlockSpec((NBLK, 1), lambda i: (i, 0)),
            pl.BlockSpec((1, H), lambda i: (0, 0)),
            pl.BlockSpec((1, H), lambda i: (0, 0)),
            pl.BlockSpec((H, H), lambda i: (0, 0)),
            pl.BlockSpec((1, H), lambda i: (0, 0)),
        ],
        out_specs=pl.BlockSpec((2, NBLK, HH), lambda i: (0, i, 0)),
        out_shape=jax.ShapeDtypeStruct((2, NP, HH), jnp.float32),
    )(node_ids.reshape(N, 1), w1, b1.reshape(1, H), w2, b2.reshape(1, H))


def _edge_mlp_body(ea_ref, w1_ref, b1_ref, w2_ref, b2_ref, o_ref):
    h = jax.nn.relu(
        jnp.dot(ea_ref[...], w1_ref[...], preferred_element_type=jnp.float32)
        + b1_ref[...])
    out = jnp.dot(h, w2_ref[...], preferred_element_type=jnp.float32) + b2_ref[...]
    o_ref[0, :, :] = out[:, :HH]
    o_ref[1, :, :] = out[:, HH:]


def _edge_mlp(edge_attr, w1, b1, w2, b2):
    return pl.pallas_call(
        _edge_mlp_body,
        grid=(E // EBLK,),
        in_specs=[
            pl.BlockSpec((EBLK, 16), lambda i: (i, 0)),
            pl.BlockSpec((16, H), lambda i: (0, 0)),
            pl.BlockSpec((1, H), lambda i: (0, 0)),
            pl.BlockSpec((H, H), lambda i: (0, 0)),
            pl.BlockSpec((1, H), lambda i: (0, 0)),
        ],
        out_specs=pl.BlockSpec((2, EBLK, HH), lambda i: (0, i, 0)),
        out_shape=jax.ShapeDtypeStruct((2, EPAD, HH), jnp.float32),
    )(edge_attr, w1, b1.reshape(1, H), w2, b2.reshape(1, H))


def _conv_mlp_body(eps_ref, x_ref, agg_ref, w1_ref, b1_ref, w2_ref, b2_ref, o_ref):
    x = jnp.concatenate([x_ref[0], x_ref[1]], axis=1)
    agg = jnp.concatenate([agg_ref[0], agg_ref[1]], axis=1)
    y = eps_ref[0, 0] * x + agg
    h = jax.nn.relu(jnp.dot(y, w1_ref[...], preferred_element_type=jnp.float32)
                    + b1_ref[...])
    out = jax.nn.relu(
        jnp.dot(h, w2_ref[...], preferred_element_type=jnp.float32) + b2_ref[...])
    o_ref[0, :, :] = out[:, :HH]
    o_ref[1, :, :] = out[:, HH:]


def _conv_mlp(eps1p, xt, agg, w1, b1, w2, b2):
    return pl.pallas_call(
        _conv_mlp_body,
        grid=(N // NBLK,),
        in_specs=[
            pl.BlockSpec(memory_space=pltpu.SMEM),
            pl.BlockSpec((2, NBLK, HH), lambda i: (0, i, 0)),
            pl.BlockSpec((2, NBLK, HH), lambda i: (0, i, 0)),
            pl.BlockSpec((H, H), lambda i: (0, 0)),
            pl.BlockSpec((1, H), lambda i: (0, 0)),
            pl.BlockSpec((H, H), lambda i: (0, 0)),
            pl.BlockSpec((1, H), lambda i: (0, 0)),
        ],
        out_specs=pl.BlockSpec((2, NBLK, HH), lambda i: (0, i, 0)),
        out_shape=jax.ShapeDtypeStruct((2, NP, HH), jnp.float32),
    )(eps1p, xt, agg, w1, b1.reshape(1, H), w2, b2.reshape(1, H))


def _conv_mlp_pool_body(eps_ref, x_ref, agg_ref, w1_ref, b1_ref, w2_ref, b2_ref,
                        batch_ref, sum_ref, cnt_ref):
    x = jnp.concatenate([x_ref[0], x_ref[1]], axis=1)
    agg = jnp.concatenate([agg_ref[0], agg_ref[1]], axis=1)
    y = eps_ref[0, 0] * x + agg
    h = jax.nn.relu(jnp.dot(y, w1_ref[...], preferred_element_type=jnp.float32)
                    + b1_ref[...])
    out = jax.nn.relu(
        jnp.dot(h, w2_ref[...], preferred_element_type=jnp.float32) + b2_ref[...])
    seg = jax.lax.broadcasted_iota(jnp.int32, (1, B), 1)
    onehot = (batch_ref[...] == seg).astype(jnp.float32)
    psum = lax.dot_general(onehot, out, (((0,), (0,)), ((), ())),
                           preferred_element_type=jnp.float32,
                           precision=lax.Precision.HIGHEST)
    pcnt = lax.dot_general(onehot, jnp.ones((NBLK, HH), jnp.float32),
                           (((0,), (0,)), ((), ())),
                           preferred_element_type=jnp.float32,
                           precision=lax.Precision.HIGHEST)

    @pl.when(pl.program_id(0) == 0)
    def _():
        sum_ref[...] = jnp.zeros_like(sum_ref)
        cnt_ref[...] = jnp.zeros_like(cnt_ref)

    sum_ref[...] += psum
    cnt_ref[...] += pcnt


def _conv_mlp_pool(eps1p, xt, agg, w1, b1, w2, b2, batch):
    return pl.pallas_call(
        _conv_mlp_pool_body,
        grid=(N // NBLK,),
        in_specs=[
            pl.BlockSpec(memory_space=pltpu.SMEM),
            pl.BlockSpec((2, NBLK, HH), lambda i: (0, i, 0)),
            pl.BlockSpec((2, NBLK, HH), lambda i: (0, i, 0)),
            pl.BlockSpec((H, H), lambda i: (0, 0)),
            pl.BlockSpec((1, H), lambda i: (0, 0)),
            pl.BlockSpec((H, H), lambda i: (0, 0)),
            pl.BlockSpec((1, H), lambda i: (0, 0)),
            pl.BlockSpec((NBLK, 1), lambda i: (i, 0)),
        ],
        out_specs=[
            pl.BlockSpec((B, H), lambda i: (0, 0)),
            pl.BlockSpec((B, HH), lambda i: (0, 0)),
        ],
        out_shape=[
            jax.ShapeDtypeStruct((B, H), jnp.float32),
            jax.ShapeDtypeStruct((B, HH), jnp.float32),
        ],
    )(eps1p, xt, agg, w1, b1.reshape(1, H), w2, b2.reshape(1, H),
      batch.reshape(N, 1))


def _head_body(ss_ref, sc_ref, gs_ref, gc_ref, dep_ref, w1s_ref, w1g_ref,
               w1d_ref, b1_ref, w2_ref, b2_ref, o_ref):
    s_emb = ss_ref[...] / jnp.maximum(sc_ref[:, :1], 1.0)
    g_emb = gs_ref[...] / jnp.maximum(gc_ref[:, :1], 1.0)
    dep = dep_ref[...].astype(jnp.bfloat16).astype(jnp.float32)
    w1d = w1d_ref[...].astype(jnp.bfloat16).astype(jnp.float32)
    z = (jnp.dot(s_emb, w1s_ref[...], preferred_element_type=jnp.float32)
         + jnp.dot(g_emb, w1g_ref[...], preferred_element_type=jnp.float32)
         + dep * w1d
         + b1_ref[...])
    h = jax.nn.relu(z)
    o_ref[...] = (jnp.dot(h, w2_ref[...], preferred_element_type=jnp.float32)
                  + b2_ref[...])


def _head(s_sum, s_cnt, g_sum, g_cnt, depth, w1, b1, w2, b2):
    out = pl.pallas_call(
        _head_body,
        out_shape=jax.ShapeDtypeStruct((B, 1), jnp.float32),
    )(s_sum, s_cnt, g_sum, g_cnt, depth.reshape(B, 1),
      w1[:H], w1[H:2 * H], w1[2 * H:].reshape(1, H), b1.reshape(1, H),
      w2, b2.reshape(1, 1))
    return out[:, 0]


def _msg_body(xt_ref, et_ref, src_ref, dst_ref, zr_ref, out_ref,
              xbA, xbB, ebA, ebB, mbA, mbB, ixA, ixB, sxA, sxB,
              acc, gsA, gsB, esA, esB, ssA, ssB, imA, imB):
    cid = lax.axis_index("c")
    sid = lax.axis_index("s")
    rbase = sid * RPT
    pltpu.sync_copy(zr_ref, acc.at[pl.ds(rbase, RPT)])
    plsc.subcore_barrier()

    ibase = sid * EPT
    ebase = cid * EPAD + sid * EPT
    off = cid * NP

    def idx_load(g, ix, im):
        pltpu.async_copy(src_ref.at[pl.ds(ibase + g * CK, CK)], ix.at[0], im)
        pltpu.async_copy(dst_ref.at[pl.ds(ibase + g * CK, CK)], ix.at[1], im)

    def idx_wait(g, ix, im):
        pltpu.make_async_copy(src_ref.at[pl.ds(ibase + g * CK, CK)], ix.at[0],
                              im).wait()
        pltpu.make_async_copy(dst_ref.at[pl.ds(ibase + g * CK, CK)], ix.at[1],
                              im).wait()
        for i in range(CK // 16):
            s = pl.ds(i * 16, 16)
            ix[0, s] = ix[0, s] + off

    def loads(g, ix, xb, eb, gs, es):
        pltpu.async_copy(xt_ref.at[ix.at[0]], xb, gs)
        pltpu.async_copy(et_ref.at[pl.ds(ebase + g * CK, CK)], eb, es)

    def proc(g, xb, eb, mb, ix, sx, gs, es, ss, im, steady):
        pltpu.make_async_copy(xt_ref.at[ix.at[0]], xb, gs).wait()
        pltpu.make_async_copy(et_ref.at[pl.ds(ebase + g * CK, CK)], eb,
                              es).wait()
        if steady:
            pltpu.make_async_copy(mb, acc.at[sx], ss).wait()
        for i in range(CK // 16):
            s = pl.ds(i * 16, 16)
            sx[s] = ix[1, s]

        @pl.when(g + 2 < NCHUNK)
        def _():
            idx_load(g + 2, ix, im)

        def rowop(r, c):
            for j in range(HH // 16):
                s = pl.ds(j * 16, 16)
                mb[r, s] = jnp.maximum(xb[r, s] + eb[r, s], 0.0)
            return c

        lax.fori_loop(0, CK, rowop, 0)
        pltpu.async_copy(mb, acc.at[sx], ss, add=True)

        @pl.when(g + 2 < NCHUNK)
        def _():
            idx_wait(g + 2, ix, im)
            loads(g + 2, ix, xb, eb, gs, es)

    idx_load(0, ixA, imA)
    idx_wait(0, ixA, imA)
    loads(0, ixA, xbA, ebA, gsA, esA)
    idx_load(1, ixB, imB)
    idx_wait(1, ixB, imB)
    loads(1, ixB, xbB, ebB, gsB, esB)

    proc(0, xbA, ebA, mbA, ixA, sxA, gsA, esA, ssA, imA, False)
    proc(1, xbB, ebB, mbB, ixB, sxB, gsB, esB, ssB, imB, False)

    def pair(i, c):
        g = 2 * i + 2
        proc(g, xbA, ebA, mbA, ixA, sxA, gsA, esA, ssA, imA, True)
        proc(g + 1, xbB, ebB, mbB, ixB, sxB, gsB, esB, ssB, imB, True)
        return c

    lax.fori_loop(0, (NCHUNK - 2) // 2, pair, 0)
    pltpu.make_async_copy(mbA, acc.at[sxA], ssA).wait()
    pltpu.make_async_copy(mbB, acc.at[sxB], ssB).wait()
    plsc.subcore_barrier()
    pltpu.sync_copy(acc.at[pl.ds(rbase, RPT)],
                    out_ref.at[cid].at[pl.ds(rbase, RPT)])


@functools.cache
def _get_msg_kernel():
    return pl.kernel(
        _msg_body,
        out_type=jax.ShapeDtypeStruct((2, NACC, HH), jnp.float32),
        mesh=plsc.VectorSubcoreMesh(core_axis_name="c", subcore_axis_name="s",
                                    num_cores=2, num_subcores=16),
        scratch_types=[
            pltpu.VMEM((CK, HH), jnp.float32),
            pltpu.VMEM((CK, HH), jnp.float32),
            pltpu.VMEM((CK, HH), jnp.float32),
            pltpu.VMEM((CK, HH), jnp.float32),
            pltpu.VMEM((CK, HH), jnp.float32),
            pltpu.VMEM((CK, HH), jnp.float32),
            pltpu.VMEM((2, CK), jnp.int32),
            pltpu.VMEM((2, CK), jnp.int32),
            pltpu.VMEM((CK,), jnp.int32),
            pltpu.VMEM((CK,), jnp.int32),
            pltpu.VMEM_SHARED((NACC, HH), jnp.float32),
            pltpu.SemaphoreType.DMA,
            pltpu.SemaphoreType.DMA,
            pltpu.SemaphoreType.DMA,
            pltpu.SemaphoreType.DMA,
            pltpu.SemaphoreType.DMA,
            pltpu.SemaphoreType.DMA,
            pltpu.SemaphoreType.DMA,
            pltpu.SemaphoreType.DMA,
        ],
    )


def _message_pass(xt, et, ei_p, zrows):
    return _get_msg_kernel()(xt.reshape(2 * NP, HH), et.reshape(2 * EPAD, HH),
                             ei_p[0], ei_p[1], zrows)


def _encode(node_ids, edge_index, e_emb, batch, params, p1, p2, zrows):
    pad = jnp.zeros((2, EPAD - E), jnp.int32).at[1].set(N)
    ei_p = jnp.concatenate([edge_index.astype(jnp.int32), pad], axis=1)
    xt = _id_mlp(node_ids, params['id_W1'], params['id_b1'],
                 params['id_W2'], params['id_b2'])
    agg1 = _message_pass(xt, e_emb, ei_p, zrows)
    eps1 = (1.0 + params[p1 + '_eps']).astype(jnp.float32).reshape(1, 1)
    xt = _conv_mlp(eps1, xt, agg1, params[p1 + '_W1'], params[p1 + '_b1'],
                   params[p1 + '_W2'], params[p1 + '_b2'])
    agg2 = _message_pass(xt, e_emb, ei_p, zrows)
    eps2 = (1.0 + params[p2 + '_eps']).astype(jnp.float32).reshape(1, 1)
    return _conv_mlp_pool(eps2, xt, agg2, params[p2 + '_W1'], params[p2 + '_b1'],
                          params[p2 + '_W2'], params[p2 + '_b2'],
                          batch.astype(jnp.int32))


def kernel(s_node_ids, s_edge_index, s_edge_attr, s_batch, depth,
           g_node_ids, g_edge_index, g_edge_attr, g_batch, params):
    zrows = jnp.zeros((RPT, HH), jnp.float32)
    s_e = _edge_mlp(s_edge_attr.astype(jnp.float32), params['edge_W1'],
                    params['edge_b1'], params['edge_W2'], params['edge_b2'])
    g_e = _edge_mlp(g_edge_attr.astype(jnp.float32), params['edge_W1'],
                    params['edge_b1'], params['edge_W2'], params['edge_b2'])
    s_sum, s_cnt = _encode(s_node_ids, s_edge_index, s_e, s_batch, params,
                           's1', 's2', zrows)
    g_sum, g_cnt = _encode(g_node_ids, g_edge_index, g_e, g_batch, params,
                           'g1', 'g2', zrows)
    return _head(s_sum, s_cnt, g_sum, g_cnt, depth.astype(jnp.float32),
                 params['reg_W1'], params['reg_b1'], params['reg_W2'],
                 params['reg_b2'])


# speedup vs baseline: 1.1506x; 1.0844x over previous
import functools

import jax
import jax.numpy as jnp
from jax import lax
from jax.experimental import pallas as pl
from jax.experimental.pallas import tpu as pltpu
from jax.experimental.pallas import tpu_sc as plsc

N = 10000
E = 160000
H = 256
HH = 128
B = 64
NP = 10240
_TWO48 = float(2 ** 48 - 1)

NBLK = 1000
EBLK = 1280

CK = 48
NCHUNK = 210
EPT = NCHUNK * CK
EPAD = 16 * EPT
NACC = 10112
RPT = NACC // 16


def _id_mlp_body(ids_ref, w1_ref, b1_ref, w2_ref, b2_ref, o_ref):
    ids = ids_ref[...].astype(jnp.float32)
    norm = jnp.clip((ids + 2.0) / _TWO48, 0.0, 1.0)
    h = jax.nn.relu(norm * w1_ref[...] + b1_ref[...])
    out = jnp.dot(h, w2_ref[...], preferred_element_type=jnp.float32) + b2_ref[...]
    o_ref[0, :, :] = out[:, :HH]
    o_ref[1, :, :] = out[:, HH:]


def _id_mlp(node_ids, w1, b1, w2, b2):
    return pl.pallas_call(
        _id_mlp_body,
        grid=(N // NBLK,),
        in_specs=[
            pl.BlockSpec((NBLK, 1), lambda i: (i, 0)),
            pl.BlockSpec((1, H), lambda i: (0, 0)),
            pl.BlockSpec((1, H), lambda i: (0, 0)),
            pl.BlockSpec((H, H), lambda i: (0, 0)),
            pl.BlockSpec((1, H), lambda i: (0, 0)),
        ],
        out_specs=pl.BlockSpec((2, NBLK, HH), lambda i: (0, i, 0)),
        out_shape=jax.ShapeDtypeStruct((2, NP, HH), jnp.float32),
    )(node_ids.reshape(N, 1), w1, b1.reshape(1, H), w2, b2.reshape(1, H))


def _edge_mlp_body(ea_ref, w1_ref, b1_ref, w2_ref, b2_ref, o_ref):
    h = jax.nn.relu(
        jnp.dot(ea_ref[...], w1_ref[...], preferred_element_type=jnp.float32)
        + b1_ref[...])
    out = jnp.dot(h, w2_ref[...], preferred_element_type=jnp.float32) + b2_ref[...]
    o_ref[0, :, :] = out[:, :HH]
    o_ref[1, :, :] = out[:, HH:]


def _edge_mlp(edge_attr, w1, b1, w2, b2):
    return pl.pallas_call(
        _edge_mlp_body,
        grid=(E // EBLK,),
        in_specs=[
            pl.BlockSpec((EBLK, 16), lambda i: (i, 0)),
            pl.BlockSpec((16, H), lambda i: (0, 0)),
            pl.BlockSpec((1, H), lambda i: (0, 0)),
            pl.BlockSpec((H, H), lambda i: (0, 0)),
            pl.BlockSpec((1, H), lambda i: (0, 0)),
        ],
        out_specs=pl.BlockSpec((2, EBLK, HH), lambda i: (0, i, 0)),
        out_shape=jax.ShapeDtypeStruct((2, EPAD, HH), jnp.float32),
    )(edge_attr, w1, b1.reshape(1, H), w2, b2.reshape(1, H))


def _conv_mlp_body(eps_ref, x_ref, agg_ref, w1_ref, b1_ref, w2_ref, b2_ref, o_ref):
    x = jnp.concatenate([x_ref[0], x_ref[1]], axis=1)
    agg = jnp.concatenate([agg_ref[0], agg_ref[1]], axis=1)
    y = eps_ref[0, 0] * x + agg
    h = jax.nn.relu(jnp.dot(y, w1_ref[...], preferred_element_type=jnp.float32)
                    + b1_ref[...])
    out = jax.nn.relu(
        jnp.dot(h, w2_ref[...], preferred_element_type=jnp.float32) + b2_ref[...])
    o_ref[0, :, :] = out[:, :HH]
    o_ref[1, :, :] = out[:, HH:]


def _conv_mlp(eps1p, xt, agg, w1, b1, w2, b2):
    return pl.pallas_call(
        _conv_mlp_body,
        grid=(N // NBLK,),
        in_specs=[
            pl.BlockSpec(memory_space=pltpu.SMEM),
            pl.BlockSpec((2, NBLK, HH), lambda i: (0, i, 0)),
            pl.BlockSpec((2, NBLK, HH), lambda i: (0, i, 0)),
            pl.BlockSpec((H, H), lambda i: (0, 0)),
            pl.BlockSpec((1, H), lambda i: (0, 0)),
            pl.BlockSpec((H, H), lambda i: (0, 0)),
            pl.BlockSpec((1, H), lambda i: (0, 0)),
        ],
        out_specs=pl.BlockSpec((2, NBLK, HH), lambda i: (0, i, 0)),
        out_shape=jax.ShapeDtypeStruct((2, NP, HH), jnp.float32),
    )(eps1p, xt, agg, w1, b1.reshape(1, H), w2, b2.reshape(1, H))


def _conv_mlp_pool_body(eps_ref, x_ref, agg_ref, w1_ref, b1_ref, w2_ref, b2_ref,
                        batch_ref, sum_ref, cnt_ref):
    x = jnp.concatenate([x_ref[0], x_ref[1]], axis=1)
    agg = jnp.concatenate([agg_ref[0], agg_ref[1]], axis=1)
    y = eps_ref[0, 0] * x + agg
    h = jax.nn.relu(jnp.dot(y, w1_ref[...], preferred_element_type=jnp.float32)
                    + b1_ref[...])
    out = jax.nn.relu(
        jnp.dot(h, w2_ref[...], preferred_element_type=jnp.float32) + b2_ref[...])
    seg = jax.lax.broadcasted_iota(jnp.int32, (1, B), 1)
    onehot = (batch_ref[...] == seg).astype(jnp.float32)
    psum = lax.dot_general(onehot, out, (((0,), (0,)), ((), ())),
                           preferred_element_type=jnp.float32,
                           precision=lax.Precision.HIGHEST)
    pcnt = lax.dot_general(onehot, jnp.ones((NBLK, HH), jnp.float32),
                           (((0,), (0,)), ((), ())),
                           preferred_element_type=jnp.float32,
                           precision=lax.Precision.HIGHEST)

    @pl.when(pl.program_id(0) == 0)
    def _():
        sum_ref[...] = jnp.zeros_like(sum_ref)
        cnt_ref[...] = jnp.zeros_like(cnt_ref)

    sum_ref[...] += psum
    cnt_ref[...] += pcnt


def _conv_mlp_pool(eps1p, xt, agg, w1, b1, w2, b2, batch):
    return pl.pallas_call(
        _conv_mlp_pool_body,
        grid=(N // NBLK,),
        in_specs=[
            pl.BlockSpec(memory_space=pltpu.SMEM),
            pl.BlockSpec((2, NBLK, HH), lambda i: (0, i, 0)),
            pl.BlockSpec((2, NBLK, HH), lambda i: (0, i, 0)),
            pl.BlockSpec((H, H), lambda i: (0, 0)),
            pl.BlockSpec((1, H), lambda i: (0, 0)),
            pl.BlockSpec((H, H), lambda i: (0, 0)),
            pl.BlockSpec((1, H), lambda i: (0, 0)),
            pl.BlockSpec((NBLK, 1), lambda i: (i, 0)),
        ],
        out_specs=[
            pl.BlockSpec((B, H), lambda i: (0, 0)),
            pl.BlockSpec((B, HH), lambda i: (0, 0)),
        ],
        out_shape=[
            jax.ShapeDtypeStruct((B, H), jnp.float32),
            jax.ShapeDtypeStruct((B, HH), jnp.float32),
        ],
    )(eps1p, xt, agg, w1, b1.reshape(1, H), w2, b2.reshape(1, H),
      batch.reshape(N, 1))


def _head_body(ss_ref, sc_ref, gs_ref, gc_ref, dep_ref, w1s_ref, w1g_ref,
               w1d_ref, b1_ref, w2_ref, b2_ref, o_ref):
    s_emb = ss_ref[...] / jnp.maximum(sc_ref[:, :1], 1.0)
    g_emb = gs_ref[...] / jnp.maximum(gc_ref[:, :1], 1.0)
    dep = dep_ref[...].astype(jnp.bfloat16).astype(jnp.float32)
    w1d = w1d_ref[...].astype(jnp.bfloat16).astype(jnp.float32)
    z = (jnp.dot(s_emb, w1s_ref[...], preferred_element_type=jnp.float32)
         + jnp.dot(g_emb, w1g_ref[...], preferred_element_type=jnp.float32)
         + dep * w1d
         + b1_ref[...])
    h = jax.nn.relu(z)
    o_ref[...] = (jnp.dot(h, w2_ref[...], preferred_element_type=jnp.float32)
                  + b2_ref[...])


def _head(s_sum, s_cnt, g_sum, g_cnt, depth, w1, b1, w2, b2):
    out = pl.pallas_call(
        _head_body,
        out_shape=jax.ShapeDtypeStruct((B, 1), jnp.float32),
    )(s_sum, s_cnt, g_sum, g_cnt, depth.reshape(B, 1),
      w1[:H], w1[H:2 * H], w1[2 * H:].reshape(1, H), b1.reshape(1, H),
      w2, b2.reshape(1, 1))
    return out[:, 0]


def _msg_body(xt_ref, et_ref, src_ref, dst_ref, zr_ref, out_ref,
              xbA, xbB, ebA, ebB, mbA, mbB, ixA, ixB, sxA, sxB,
              acc, gsA, gsB, esA, esB, ssA, ssB, imA, imB):
    cid = lax.axis_index("c")
    sid = lax.axis_index("s")
    rbase = sid * RPT
    pltpu.sync_copy(zr_ref, acc.at[pl.ds(rbase, RPT)])
    plsc.subcore_barrier()

    ibase = sid * EPT
    ebase = cid * EPAD + sid * EPT
    off = cid * NP

    def idx_load(g, ix, im):
        pltpu.async_copy(src_ref.at[pl.ds(ibase + g * CK, CK)], ix.at[0], im)
        pltpu.async_copy(dst_ref.at[pl.ds(ibase + g * CK, CK)], ix.at[1], im)

    def idx_wait(g, ix, im):
        pltpu.make_async_copy(src_ref.at[pl.ds(ibase + g * CK, CK)], ix.at[0],
                              im).wait()
        pltpu.make_async_copy(dst_ref.at[pl.ds(ibase + g * CK, CK)], ix.at[1],
                              im).wait()
        for i in range(CK // 16):
            s = pl.ds(i * 16, 16)
            ix[0, s] = ix[0, s] + off

    def loads(g, ix, xb, eb, gs, es):
        pltpu.async_copy(xt_ref.at[ix.at[0]], xb, gs)
        pltpu.async_copy(et_ref.at[pl.ds(ebase + g * CK, CK)], eb, es)

    def proc(g, xb, eb, mb, ix, sx, gs, es, ss, im, steady):
        pltpu.make_async_copy(xt_ref.at[ix.at[0]], xb, gs).wait()
        pltpu.make_async_copy(et_ref.at[pl.ds(ebase + g * CK, CK)], eb,
                              es).wait()
        if steady:
            pltpu.make_async_copy(mb, acc.at[sx], ss).wait()
        for i in range(CK // 16):
            s = pl.ds(i * 16, 16)
            sx[s] = ix[1, s]

        @pl.when(g + 2 < NCHUNK)
        def _():
            idx_load(g + 2, ix, im)

        def rowop(r, c):
            for j in range(HH // 16):
                s = pl.ds(j * 16, 16)
                mb[r, s] = jnp.maximum(xb[r, s] + eb[r, s], 0.0)
            return c

        lax.fori_loop(0, CK, rowop, 0)
        pltpu.async_copy(mb, acc.at[sx], ss, add=True)

        @pl.when(g + 2 < NCHUNK)
        def _():
            idx_wait(g + 2, ix, im)
            loads(g + 2, ix, xb, eb, gs, es)

    idx_load(0, ixA, imA)
    idx_wait(0, ixA, imA)
    loads(0, ixA, xbA, ebA, gsA, esA)
    idx_load(1, ixB, imB)
    idx_wait(1, ixB, imB)
    loads(1, ixB, xbB, ebB, gsB, esB)

    proc(0, xbA, ebA, mbA, ixA, sxA, gsA, esA, ssA, imA, False)
    proc(1, xbB, ebB, mbB, ixB, sxB, gsB, esB, ssB, imB, False)

    def pair(i, c):
        g = 2 * i + 2
        proc(g, xbA, ebA, mbA, ixA, sxA, gsA, esA, ssA, imA, True)
        proc(g + 1, xbB, ebB, mbB, ixB, sxB, gsB, esB, ssB, imB, True)
        return c

    lax.fori_loop(0, (NCHUNK - 2) // 2, pair, 0)
    pltpu.make_async_copy(mbA, acc.at[sxA], ssA).wait()
    pltpu.make_async_copy(mbB, acc.at[sxB], ssB).wait()
    plsc.subcore_barrier()
    pltpu.sync_copy(acc.at[pl.ds(rbase, RPT)],
                    out_ref.at[cid].at[pl.ds(rbase, RPT)])


@functools.cache
def _get_msg_kernel():
    return pl.kernel(
        _msg_body,
        out_type=jax.ShapeDtypeStruct((2, NACC, HH), jnp.float32),
        mesh=plsc.VectorSubcoreMesh(core_axis_name="c", subcore_axis_name="s",
                                    num_cores=2, num_subcores=16),
        scratch_types=[
            pltpu.VMEM((CK, HH), jnp.float32),
            pltpu.VMEM((CK, HH), jnp.float32),
            pltpu.VMEM((CK, HH), jnp.float32),
            pltpu.VMEM((CK, HH), jnp.float32),
            pltpu.VMEM((CK, HH), jnp.float32),
            pltpu.VMEM((CK, HH), jnp.float32),
            pltpu.VMEM((2, CK), jnp.int32),
            pltpu.VMEM((2, CK), jnp.int32),
            pltpu.VMEM((CK,), jnp.int32),
            pltpu.VMEM((CK,), jnp.int32),
            pltpu.VMEM_SHARED((NACC, HH), jnp.float32),
            pltpu.SemaphoreType.DMA,
            pltpu.SemaphoreType.DMA,
            pltpu.SemaphoreType.DMA,
            pltpu.SemaphoreType.DMA,
            pltpu.SemaphoreType.DMA,
            pltpu.SemaphoreType.DMA,
            pltpu.SemaphoreType.DMA,
            pltpu.SemaphoreType.DMA,
        ],
    )


def _message_pass(xt, et, ei_p, zrows):
    return _get_msg_kernel()(xt.reshape(2 * NP, HH), et.reshape(2 * EPAD, HH),
                             ei_p[0], ei_p[1], zrows)


def _encode(node_ids, edge_index, e_emb, batch, params, p1, p2, zrows):
    pad = jnp.zeros((2, EPAD - E), jnp.int32).at[1].set(N)
    ei_p = jnp.concatenate([edge_index.astype(jnp.int32), pad], axis=1)
    xt = _id_mlp(node_ids, params['id_W1'], params['id_b1'],
                 params['id_W2'], params['id_b2'])
    agg1 = _message_pass(xt, e_emb, ei_p, zrows)
    eps1 = (1.0 + params[p1 + '_eps']).astype(jnp.float32).reshape(1, 1)
    xt = _conv_mlp(eps1, xt, agg1, params[p1 + '_W1'], params[p1 + '_b1'],
                   params[p1 + '_W2'], params[p1 + '_b2'])
    agg2 = _message_pass(xt, e_emb, ei_p, zrows)
    eps2 = (1.0 + params[p2 + '_eps']).astype(jnp.float32).reshape(1, 1)
    return _conv_mlp_pool(eps2, xt, agg2, params[p2 + '_W1'], params[p2 + '_b1'],
                          params[p2 + '_W2'], params[p2 + '_b2'],
                          batch.astype(jnp.int32))


def kernel(s_node_ids, s_edge_index, s_edge_attr, s_batch, depth,
           g_node_ids, g_edge_index, g_edge_attr, g_batch, params):
    zrows = jnp.zeros((RPT, HH), jnp.float32)
    s_e = _edge_mlp(s_edge_attr.astype(jnp.float32), params['edge_W1'],
                    params['edge_b1'], params['edge_W2'], params['edge_b2'])
    g_e = _edge_mlp(g_edge_attr.astype(jnp.float32), params['edge_W1'],
                    params['edge_b1'], params['edge_W2'], params['edge_b2'])
    s_sum, s_cnt = _encode(s_node_ids, s_edge_index, s_e, s_batch, params,
                           's1', 's2', zrows)
    g_sum, g_cnt = _encode(g_node_ids, g_edge_index, g_e, g_batch, params,
                           'g1', 'g2', zrows)
    return _head(s_sum, s_cnt, g_sum, g_cnt, depth.astype(jnp.float32),
                 params['reg_W1'], params['reg_b1'], params['reg_W2'],
                 params['reg_b2'])


# speedup vs baseline: 1.1816x; 1.0269x over previous
import functools

import jax
import jax.numpy as jnp
from jax import lax
from jax.experimental import pallas as pl
from jax.experimental.pallas import tpu as pltpu
from jax.experimental.pallas import tpu_sc as plsc

N = 10000
E = 160000
H = 256
HH = 128
B = 64
NP = 10240
_TWO48 = float(2 ** 48 - 1)

NBLK = 2000
EBLK = 3200

CK = 48
NCHUNK = 210
EPT = NCHUNK * CK
EPAD = 16 * EPT
NACC = 10112
RPT = NACC // 16


def _id_mlp_body(ids_ref, w1_ref, b1_ref, w2_ref, b2_ref, o_ref):
    ids = ids_ref[...].astype(jnp.float32)
    norm = jnp.clip((ids + 2.0) / _TWO48, 0.0, 1.0)
    h = jax.nn.relu(norm * w1_ref[...] + b1_ref[...])
    out = jnp.dot(h, w2_ref[...], preferred_element_type=jnp.float32) + b2_ref[...]
    o_ref[0, :, :] = out[:, :HH]
    o_ref[1, :, :] = out[:, HH:]


def _id_mlp(node_ids, w1, b1, w2, b2):
    return pl.pallas_call(
        _id_mlp_body,
        grid=(N // NBLK,),
        in_specs=[
            pl.BlockSpec((NBLK, 1), lambda i: (i, 0)),
            pl.BlockSpec((1, H), lambda i: (0, 0)),
            pl.BlockSpec((1, H), lambda i: (0, 0)),
            pl.BlockSpec((H, H), lambda i: (0, 0)),
            pl.BlockSpec((1, H), lambda i: (0, 0)),
        ],
        out_specs=pl.BlockSpec((2, NBLK, HH), lambda i: (0, i, 0)),
        out_shape=jax.ShapeDtypeStruct((2, NP, HH), jnp.float32),
    )(node_ids.reshape(N, 1), w1, b1.reshape(1, H), w2, b2.reshape(1, H))


def _edge_mlp_body(ea_ref, w1_ref, b1_ref, w2_ref, b2_ref, o_ref):
    h = jax.nn.relu(
        jnp.dot(ea_ref[...], w1_ref[...], preferred_element_type=jnp.float32)
        + b1_ref[...])
    out = jnp.dot(h, w2_ref[...], preferred_element_type=jnp.float32) + b2_ref[...]
    o_ref[0, :, :] = out[:, :HH]
    o_ref[1, :, :] = out[:, HH:]


def _edge_mlp(edge_attr, w1, b1, w2, b2):
    return pl.pallas_call(
        _edge_mlp_body,
        grid=(E // EBLK,),
        in_specs=[
            pl.BlockSpec((EBLK, 16), lambda i: (i, 0)),
            pl.BlockSpec((16, H), lambda i: (0, 0)),
            pl.BlockSpec((1, H), lambda i: (0, 0)),
            pl.BlockSpec((H, H), lambda i: (0, 0)),
            pl.BlockSpec((1, H), lambda i: (0, 0)),
        ],
        out_specs=pl.BlockSpec((2, EBLK, HH), lambda i: (0, i, 0)),
        out_shape=jax.ShapeDtypeStruct((2, EPAD, HH), jnp.float32),
    )(edge_attr, w1, b1.reshape(1, H), w2, b2.reshape(1, H))


def _conv_mlp_body(eps_ref, x_ref, agg_ref, w1_ref, b1_ref, w2_ref, b2_ref, o_ref):
    x = jnp.concatenate([x_ref[0], x_ref[1]], axis=1)
    agg = jnp.concatenate([agg_ref[0], agg_ref[1]], axis=1)
    y = eps_ref[0, 0] * x + agg
    h = jax.nn.relu(jnp.dot(y, w1_ref[...], preferred_element_type=jnp.float32)
                    + b1_ref[...])
    out = jax.nn.relu(
        jnp.dot(h, w2_ref[...], preferred_element_type=jnp.float32) + b2_ref[...])
    o_ref[0, :, :] = out[:, :HH]
    o_ref[1, :, :] = out[:, HH:]


def _conv_mlp(eps1p, xt, agg, w1, b1, w2, b2):
    return pl.pallas_call(
        _conv_mlp_body,
        grid=(N // NBLK,),
        in_specs=[
            pl.BlockSpec(memory_space=pltpu.SMEM),
            pl.BlockSpec((2, NBLK, HH), lambda i: (0, i, 0)),
            pl.BlockSpec((2, NBLK, HH), lambda i: (0, i, 0)),
            pl.BlockSpec((H, H), lambda i: (0, 0)),
            pl.BlockSpec((1, H), lambda i: (0, 0)),
            pl.BlockSpec((H, H), lambda i: (0, 0)),
            pl.BlockSpec((1, H), lambda i: (0, 0)),
        ],
        out_specs=pl.BlockSpec((2, NBLK, HH), lambda i: (0, i, 0)),
        out_shape=jax.ShapeDtypeStruct((2, NP, HH), jnp.float32),
    )(eps1p, xt, agg, w1, b1.reshape(1, H), w2, b2.reshape(1, H))


def _conv_mlp_pool_body(eps_ref, x_ref, agg_ref, w1_ref, b1_ref, w2_ref, b2_ref,
                        batch_ref, sum_ref, cnt_ref):
    x = jnp.concatenate([x_ref[0], x_ref[1]], axis=1)
    agg = jnp.concatenate([agg_ref[0], agg_ref[1]], axis=1)
    y = eps_ref[0, 0] * x + agg
    h = jax.nn.relu(jnp.dot(y, w1_ref[...], preferred_element_type=jnp.float32)
                    + b1_ref[...])
    out = jax.nn.relu(
        jnp.dot(h, w2_ref[...], preferred_element_type=jnp.float32) + b2_ref[...])
    seg = jax.lax.broadcasted_iota(jnp.int32, (1, B), 1)
    onehot = (batch_ref[...] == seg).astype(jnp.float32)
    psum = lax.dot_general(onehot, out, (((0,), (0,)), ((), ())),
                           preferred_element_type=jnp.float32,
                           precision=lax.Precision.HIGHEST)
    pcnt = lax.dot_general(onehot, jnp.ones((NBLK, HH), jnp.float32),
                           (((0,), (0,)), ((), ())),
                           preferred_element_type=jnp.float32,
                           precision=lax.Precision.HIGHEST)

    @pl.when(pl.program_id(0) == 0)
    def _():
        sum_ref[...] = jnp.zeros_like(sum_ref)
        cnt_ref[...] = jnp.zeros_like(cnt_ref)

    sum_ref[...] += psum
    cnt_ref[...] += pcnt


def _conv_mlp_pool(eps1p, xt, agg, w1, b1, w2, b2, batch):
    return pl.pallas_call(
        _conv_mlp_pool_body,
        grid=(N // NBLK,),
        in_specs=[
            pl.BlockSpec(memory_space=pltpu.SMEM),
            pl.BlockSpec((2, NBLK, HH), lambda i: (0, i, 0)),
            pl.BlockSpec((2, NBLK, HH), lambda i: (0, i, 0)),
            pl.BlockSpec((H, H), lambda i: (0, 0)),
            pl.BlockSpec((1, H), lambda i: (0, 0)),
            pl.BlockSpec((H, H), lambda i: (0, 0)),
            pl.BlockSpec((1, H), lambda i: (0, 0)),
            pl.BlockSpec((NBLK, 1), lambda i: (i, 0)),
        ],
        out_specs=[
            pl.BlockSpec((B, H), lambda i: (0, 0)),
            pl.BlockSpec((B, HH), lambda i: (0, 0)),
        ],
        out_shape=[
            jax.ShapeDtypeStruct((B, H), jnp.float32),
            jax.ShapeDtypeStruct((B, HH), jnp.float32),
        ],
    )(eps1p, xt, agg, w1, b1.reshape(1, H), w2, b2.reshape(1, H),
      batch.reshape(N, 1))


def _head_body(ss_ref, sc_ref, gs_ref, gc_ref, dep_ref, w1s_ref, w1g_ref,
               w1d_ref, b1_ref, w2_ref, b2_ref, o_ref):
    s_emb = ss_ref[...] / jnp.maximum(sc_ref[:, :1], 1.0)
    g_emb = gs_ref[...] / jnp.maximum(gc_ref[:, :1], 1.0)
    dep = dep_ref[...].astype(jnp.bfloat16).astype(jnp.float32)
    w1d = w1d_ref[...].astype(jnp.bfloat16).astype(jnp.float32)
    z = (jnp.dot(s_emb, w1s_ref[...], preferred_element_type=jnp.float32)
         + jnp.dot(g_emb, w1g_ref[...], preferred_element_type=jnp.float32)
         + dep * w1d
         + b1_ref[...])
    h = jax.nn.relu(z)
    o_ref[...] = (jnp.dot(h, w2_ref[...], preferred_element_type=jnp.float32)
                  + b2_ref[...])


def _head(s_sum, s_cnt, g_sum, g_cnt, depth, w1, b1, w2, b2):
    out = pl.pallas_call(
        _head_body,
        out_shape=jax.ShapeDtypeStruct((B, 1), jnp.float32),
    )(s_sum, s_cnt, g_sum, g_cnt, depth.reshape(B, 1),
      w1[:H], w1[H:2 * H], w1[2 * H:].reshape(1, H), b1.reshape(1, H),
      w2, b2.reshape(1, 1))
    return out[:, 0]


def _msg_body(xt_ref, et_ref, src_ref, dst_ref, zr_ref, out_ref,
              xbA, xbB, ebA, ebB, mbA, mbB, ixA, ixB, sxA, sxB,
              acc, gsA, gsB, esA, esB, ssA, ssB, imA, imB):
    cid = lax.axis_index("c")
    sid = lax.axis_index("s")
    rbase = sid * RPT
    pltpu.sync_copy(zr_ref, acc.at[pl.ds(rbase, RPT)])
    plsc.subcore_barrier()

    ibase = sid * EPT
    ebase = cid * EPAD + sid * EPT
    off = cid * NP

    def idx_load(g, ix, im):
        pltpu.async_copy(src_ref.at[pl.ds(ibase + g * CK, CK)], ix.at[0], im)
        pltpu.async_copy(dst_ref.at[pl.ds(ibase + g * CK, CK)], ix.at[1], im)

    def idx_wait(g, ix, im):
        pltpu.make_async_copy(src_ref.at[pl.ds(ibase + g * CK, CK)], ix.at[0],
                              im).wait()
        pltpu.make_async_copy(dst_ref.at[pl.ds(ibase + g * CK, CK)], ix.at[1],
                              im).wait()
        for i in range(CK // 16):
            s = pl.ds(i * 16, 16)
            ix[0, s] = ix[0, s] + off

    def loads(g, ix, xb, eb, gs, es):
        pltpu.async_copy(xt_ref.at[ix.at[0]], xb, gs)
        pltpu.async_copy(et_ref.at[pl.ds(ebase + g * CK, CK)], eb, es)

    def proc(g, xb, eb, mb, ix, sx, gs, es, ss, im, steady):
        pltpu.make_async_copy(xt_ref.at[ix.at[0]], xb, gs).wait()
        pltpu.make_async_copy(et_ref.at[pl.ds(ebase + g * CK, CK)], eb,
                              es).wait()
        if steady:
            pltpu.make_async_copy(mb, acc.at[sx], ss).wait()
        for i in range(CK // 16):
            s = pl.ds(i * 16, 16)
            sx[s] = ix[1, s]

        @pl.when(g + 2 < NCHUNK)
        def _():
            idx_load(g + 2, ix, im)

        def rowop(r, c):
            for j in range(HH // 16):
                s = pl.ds(j * 16, 16)
                mb[r, s] = jnp.maximum(xb[r, s] + eb[r, s], 0.0)
            return c

        lax.fori_loop(0, CK, rowop, 0)
        pltpu.async_copy(mb, acc.at[sx], ss, add=True)

        @pl.when(g + 2 < NCHUNK)
        def _():
            idx_wait(g + 2, ix, im)
            loads(g + 2, ix, xb, eb, gs, es)

    idx_load(0, ixA, imA)
    idx_wait(0, ixA, imA)
    loads(0, ixA, xbA, ebA, gsA, esA)
    idx_load(1, ixB, imB)
    idx_wait(1, ixB, imB)
    loads(1, ixB, xbB, ebB, gsB, esB)

    proc(0, xbA, ebA, mbA, ixA, sxA, gsA, esA, ssA, imA, False)
    proc(1, xbB, ebB, mbB, ixB, sxB, gsB, esB, ssB, imB, False)

    def pair(i, c):
        g = 2 * i + 2
        proc(g, xbA, ebA, mbA, ixA, sxA, gsA, esA, ssA, imA, True)
        proc(g + 1, xbB, ebB, mbB, ixB, sxB, gsB, esB, ssB, imB, True)
        return c

    lax.fori_loop(0, (NCHUNK - 2) // 2, pair, 0)
    pltpu.make_async_copy(mbA, acc.at[sxA], ssA).wait()
    pltpu.make_async_copy(mbB, acc.at[sxB], ssB).wait()
    plsc.subcore_barrier()
    pltpu.sync_copy(acc.at[pl.ds(rbase, RPT)],
                    out_ref.at[cid].at[pl.ds(rbase, RPT)])


@functools.cache
def _get_msg_kernel():
    return pl.kernel(
        _msg_body,
        out_type=jax.ShapeDtypeStruct((2, NACC, HH), jnp.float32),
        mesh=plsc.VectorSubcoreMesh(core_axis_name="c", subcore_axis_name="s",
                                    num_cores=2, num_subcores=16),
        scratch_types=[
            pltpu.VMEM((CK, HH), jnp.float32),
            pltpu.VMEM((CK, HH), jnp.float32),
            pltpu.VMEM((CK, HH), jnp.float32),
            pltpu.VMEM((CK, HH), jnp.float32),
            pltpu.VMEM((CK, HH), jnp.float32),
            pltpu.VMEM((CK, HH), jnp.float32),
            pltpu.VMEM((2, CK), jnp.int32),
            pltpu.VMEM((2, CK), jnp.int32),
            pltpu.VMEM((CK,), jnp.int32),
            pltpu.VMEM((CK,), jnp.int32),
            pltpu.VMEM_SHARED((NACC, HH), jnp.float32),
            pltpu.SemaphoreType.DMA,
            pltpu.SemaphoreType.DMA,
            pltpu.SemaphoreType.DMA,
            pltpu.SemaphoreType.DMA,
            pltpu.SemaphoreType.DMA,
            pltpu.SemaphoreType.DMA,
            pltpu.SemaphoreType.DMA,
            pltpu.SemaphoreType.DMA,
        ],
    )


def _message_pass(xt, et, ei_p, zrows):
    return _get_msg_kernel()(xt.reshape(2 * NP, HH), et.reshape(2 * EPAD, HH),
                             ei_p[0], ei_p[1], zrows)


def _encode(node_ids, edge_index, e_emb, batch, params, p1, p2, zrows):
    pad = jnp.zeros((2, EPAD - E), jnp.int32).at[1].set(N)
    ei_p = jnp.concatenate([edge_index.astype(jnp.int32), pad], axis=1)
    xt = _id_mlp(node_ids, params['id_W1'], params['id_b1'],
                 params['id_W2'], params['id_b2'])
    agg1 = _message_pass(xt, e_emb, ei_p, zrows)
    eps1 = (1.0 + params[p1 + '_eps']).astype(jnp.float32).reshape(1, 1)
    xt = _conv_mlp(eps1, xt, agg1, params[p1 + '_W1'], params[p1 + '_b1'],
                   params[p1 + '_W2'], params[p1 + '_b2'])
    agg2 = _message_pass(xt, e_emb, ei_p, zrows)
    eps2 = (1.0 + params[p2 + '_eps']).astype(jnp.float32).reshape(1, 1)
    return _conv_mlp_pool(eps2, xt, agg2, params[p2 + '_W1'], params[p2 + '_b1'],
                          params[p2 + '_W2'], params[p2 + '_b2'],
                          batch.astype(jnp.int32))


def kernel(s_node_ids, s_edge_index, s_edge_attr, s_batch, depth,
           g_node_ids, g_edge_index, g_edge_attr, g_batch, params):
    zrows = jnp.zeros((RPT, HH), jnp.float32)
    s_e = _edge_mlp(s_edge_attr.astype(jnp.float32), params['edge_W1'],
                    params['edge_b1'], params['edge_W2'], params['edge_b2'])
    g_e = _edge_mlp(g_edge_attr.astype(jnp.float32), params['edge_W1'],
                    params['edge_b1'], params['edge_W2'], params['edge_b2'])
    s_sum, s_cnt = _encode(s_node_ids, s_edge_index, s_e, s_batch, params,
                           's1', 's2', zrows)
    g_sum, g_cnt = _encode(g_node_ids, g_edge_index, g_e, g_batch, params,
                           'g1', 'g2', zrows)
    return _head(s_sum, s_cnt, g_sum, g_cnt, depth.astype(jnp.float32),
                 params['reg_W1'], params['reg_b1'], params['reg_W2'],
                 params['reg_b2'])
